# Initial kernel scaffold; baseline (speedup 1.0000x reference)
import jax
import jax.numpy as jnp
from jax.experimental import pallas as pl


def kernel(X_n, edge_index, edge_attr, PE, snorm, batch, sketch_features, params):
    raise NotImplementedError("write your pallas kernel here")


# speedup vs baseline: 4.5809x; 4.5809x over previous
import functools

import jax
import jax.numpy as jnp
from jax import lax
from jax.experimental import pallas as pl
from jax.experimental.pallas import tpu as pltpu
from jax.experimental.pallas import tpu_sc as plsc

N = 10000
E = 320000
D = 128
NG = 128
NA = 4
NPAD = 10240
EPAD = 327680
NTILES = 32
EPT = EPAD // NTILES
K = 128
NCH = EPT // K
BLK = 512
NBLK = NPAD // BLK
ROWS_PT = NPAD // 16

_f32 = jnp.float32


def _valid_col(i):
    row = lax.broadcasted_iota(jnp.int32, (BLK, 1), 0) + i * BLK
    return (row < N).astype(_f32)


def _enc_body(x_ref, pe_ref, w_in_ref, w_pe_ref, b_in_ref, b_pe_ref, emb_ref,
              h_ref, aug_ref):
    i = pl.program_id(0)
    h = (jnp.dot(x_ref[...], w_in_ref[...], preferred_element_type=_f32)
         + jnp.dot(pe_ref[...], w_pe_ref[...], preferred_element_type=_f32)
         + b_in_ref[...] + b_pe_ref[...])
    h_ref[...] = h
    valid = _valid_col(i)
    for a in range(NA):
        aug_ref[a] = jnp.maximum(h + emb_ref[a], 0.0) * valid


def _encoder(x, pe, w_in, w_pe, b_in, b_pe, emb0):
    return pl.pallas_call(
        _enc_body,
        grid=(NBLK,),
        in_specs=[
            pl.BlockSpec((BLK, D), lambda i: (i, 0)),
            pl.BlockSpec((BLK, D), lambda i: (i, 0)),
            pl.BlockSpec((D, D), lambda i: (0, 0)),
            pl.BlockSpec((D, D), lambda i: (0, 0)),
            pl.BlockSpec((D,), lambda i: (0,)),
            pl.BlockSpec((D,), lambda i: (0,)),
            pl.BlockSpec((NA, D), lambda i: (0, 0)),
        ],
        out_specs=[
            pl.BlockSpec((BLK, D), lambda i: (i, 0)),
            pl.BlockSpec((NA, BLK, D), lambda i: (0, i, 0)),
        ],
        out_shape=[
            jax.ShapeDtypeStruct((NPAD, D), _f32),
            jax.ShapeDtypeStruct((NA, NPAD, D), _f32),
        ],
    )(x, pe, w_in, w_pe, b_in, b_pe, emb0)


def _layer_body(h_ref, agg_ref, eps_ref, w1_ref, b1_ref, w2_ref, b2_ref,
                emb_ref, h_ref_out, aug_ref):
    i = pl.program_id(0)
    z = (1.0 + eps_ref[0]) * h_ref[...] + agg_ref[0] + agg_ref[1]
    u = jnp.maximum(
        jnp.dot(z, w1_ref[...], preferred_element_type=_f32) + b1_ref[...], 0.0)
    h2 = jnp.dot(u, w2_ref[...], preferred_element_type=_f32) + b2_ref[...]
    h_ref_out[...] = h2
    valid = _valid_col(i)
    for a in range(NA):
        aug_ref[a] = jnp.maximum(h2 + emb_ref[a], 0.0) * valid


def _layer(h, agg2, eps, w1, b1, w2, b2, emb_next):
    return pl.pallas_call(
        _layer_body,
        grid=(NBLK,),
        in_specs=[
            pl.BlockSpec((BLK, D), lambda i: (i, 0)),
            pl.BlockSpec((2, BLK, D), lambda i: (0, i, 0)),
            pl.BlockSpec(memory_space=pltpu.SMEM),
            pl.BlockSpec((D, D), lambda i: (0, 0)),
            pl.BlockSpec((D,), lambda i: (0,)),
            pl.BlockSpec((D, D), lambda i: (0, 0)),
            pl.BlockSpec((D,), lambda i: (0,)),
            pl.BlockSpec((NA, D), lambda i: (0, 0)),
        ],
        out_specs=[
            pl.BlockSpec((BLK, D), lambda i: (i, 0)),
            pl.BlockSpec((NA, BLK, D), lambda i: (0, i, 0)),
        ],
        out_shape=[
            jax.ShapeDtypeStruct((NPAD, D), _f32),
            jax.ShapeDtypeStruct((NA, NPAD, D), _f32),
        ],
    )(h, agg2, eps, w1, b1, w2, b2, emb_next)


def _final_body(h_ref, agg_ref, eps_ref, w1_ref, b1_ref, w2_ref, b2_ref,
                batch_ref, wf1_ref, bf1_ref, wf2_ref, bf2_ref,
                psum_ref, cnt_ref, y_ref):
    step = pl.program_id(0)

    @pl.when(step == 0)
    def _init():
        psum_ref[...] = jnp.zeros((NG, D), _f32)
        cnt_ref[...] = jnp.zeros((NG, D), _f32)

    z = (1.0 + eps_ref[0]) * h_ref[...] + agg_ref[0] + agg_ref[1]
    u = jnp.maximum(
        jnp.dot(z, w1_ref[...], preferred_element_type=_f32) + b1_ref[...], 0.0)
    h3 = jnp.dot(u, w2_ref[...], preferred_element_type=_f32) + b2_ref[...]

    bvec = batch_ref[0, 0, :]
    gi = lax.broadcasted_iota(jnp.int32, (NG, BLK), 0)
    mask = (gi == bvec[None, :]).astype(_f32)
    psum_ref[...] += jnp.dot(mask, h3, preferred_element_type=_f32)
    cnt_ref[...] += jnp.dot(mask, jnp.ones((BLK, D), _f32),
                            preferred_element_type=_f32)

    @pl.when(step == NBLK - 1)
    def _head():
        pooled = psum_ref[...] / jnp.maximum(cnt_ref[...], 1.0)
        t = jnp.maximum(
            jnp.dot(pooled, wf1_ref[...], preferred_element_type=_f32)
            + bf1_ref[...], 0.0)
        y_ref[...] = (jnp.dot(t, wf2_ref[...], preferred_element_type=_f32)
                      + bf2_ref[0])


def _final(h, agg2, eps, w1, b1, w2, b2, batch3, wf1, bf1, wf2p, bf2):
    outs = pl.pallas_call(
        _final_body,
        grid=(NBLK,),
        in_specs=[
            pl.BlockSpec((BLK, D), lambda i: (i, 0)),
            pl.BlockSpec((2, BLK, D), lambda i: (0, i, 0)),
            pl.BlockSpec(memory_space=pltpu.SMEM),
            pl.BlockSpec((D, D), lambda i: (0, 0)),
            pl.BlockSpec((D,), lambda i: (0,)),
            pl.BlockSpec((D, D), lambda i: (0, 0)),
            pl.BlockSpec((D,), lambda i: (0,)),
            pl.BlockSpec((1, 1, BLK), lambda i: (i, 0, 0)),
            pl.BlockSpec((D, D), lambda i: (0, 0)),
            pl.BlockSpec((D,), lambda i: (0,)),
            pl.BlockSpec((D, D), lambda i: (0, 0)),
            pl.BlockSpec(memory_space=pltpu.SMEM),
        ],
        out_specs=[
            pl.BlockSpec((NG, D), lambda i: (0, 0)),
            pl.BlockSpec((NG, D), lambda i: (0, 0)),
            pl.BlockSpec((NG, D), lambda i: (0, 0)),
        ],
        out_shape=[
            jax.ShapeDtypeStruct((NG, D), _f32),
            jax.ShapeDtypeStruct((NG, D), _f32),
            jax.ShapeDtypeStruct((NG, D), _f32),
        ],
    )(h, agg2, eps, w1, b1, w2, b2, batch3, wf1, bf1, wf2p, bf2)
    return outs[2]


_SC_MESH = plsc.VectorSubcoreMesh(core_axis_name="c", subcore_axis_name="s")


@functools.partial(
    pl.kernel,
    out_type=jax.ShapeDtypeStruct((2, NPAD, D), _f32),
    mesh=_SC_MESH,
    scratch_types=[
        pltpu.VMEM((3, K), jnp.int32),
        pltpu.VMEM((3, K), jnp.int32),
        pltpu.VMEM((K,), jnp.int32),
        pltpu.VMEM((K,), jnp.int32),
        pltpu.VMEM((K,), jnp.int32),
        pltpu.VMEM((K,), jnp.int32),
        pltpu.VMEM((K, D), _f32),
        pltpu.VMEM((K, D), _f32),
        pltpu.VMEM_SHARED((NPAD, D), _f32),
        pltpu.SemaphoreType.DMA,
        pltpu.SemaphoreType.DMA,
        pltpu.SemaphoreType.DMA,
        pltpu.SemaphoreType.DMA,
        pltpu.SemaphoreType.DMA,
        pltpu.SemaphoreType.DMA,
    ],
)
def _mp_kernel(aug_hbm, edges_hbm, zeros_hbm, out_hbm,
               e0, e1, g0, g1, d0, d1, r0, r1, agg_sh,
               es0, es1, gs0, gs1, ss0, ss1):
    cid = lax.axis_index("c")
    sid = lax.axis_index("s")
    tid = cid * 16 + sid
    cbase = tid * NCH

    ebuf = (e0, e1)
    gbuf = (g0, g1)
    dbuf = (d0, d1)
    rbuf = (r0, r1)
    esem = (es0, es1)
    gsem = (gs0, gs1)
    ssem = (ss0, ss1)

    row0 = pl.multiple_of(sid * ROWS_PT, ROWS_PT)
    pltpu.sync_copy(zeros_hbm, agg_sh.at[pl.ds(row0, ROWS_PT)])
    plsc.subcore_barrier()

    def start_edge(c, b):
        pltpu.async_copy(edges_hbm.at[cbase + c], ebuf[b], esem[b])

    def wait_edge(b):
        pltpu.make_async_copy(edges_hbm.at[cbase], ebuf[b], esem[b]).wait()

    def compute_idx(b):
        for j in range(K // 16):
            s16 = ebuf[b][0, pl.ds(j * 16, 16)]
            a16 = ebuf[b][1, pl.ds(j * 16, 16)]
            gbuf[b][pl.ds(j * 16, 16)] = a16 * NPAD + s16
            dbuf[b][pl.ds(j * 16, 16)] = ebuf[b][2, pl.ds(j * 16, 16)]

    def start_gather(b):
        pltpu.async_copy(aug_hbm.at[gbuf[b]], rbuf[b], gsem[b])

    def wait_gather(b):
        pltpu.make_async_copy(aug_hbm.at[gbuf[b]], rbuf[b], gsem[b]).wait()

    def start_scatter(b):
        pltpu.async_copy(rbuf[b], agg_sh.at[dbuf[b]], ssem[b], add=True)

    def wait_scatter(b):
        pltpu.make_async_copy(rbuf[b], agg_sh.at[dbuf[b]], ssem[b]).wait()

    start_edge(0, 0)
    start_edge(1, 1)
    wait_edge(0)
    compute_idx(0)
    start_edge(2, 0)
    start_gather(0)

    def loop_body(t, carry):
        wait_edge(1)

        @pl.when(t > 0)
        def _():
            wait_scatter(1)
        compute_idx(1)

        @pl.when(t < NCH // 2 - 1)
        def _():
            start_edge(2 * t + 3, 1)
        start_gather(1)

        wait_gather(0)
        start_scatter(0)

        @pl.when(t < NCH // 2 - 1)
        def _():
            wait_edge(0)
            wait_scatter(0)
            compute_idx(0)

            @pl.when(t < NCH // 2 - 2)
            def _():
                start_edge(2 * t + 4, 0)
            start_gather(0)

        wait_gather(1)
        start_scatter(1)
        return carry

    lax.fori_loop(0, NCH // 2, loop_body, 0)
    wait_scatter(0)
    wait_scatter(1)

    plsc.subcore_barrier()
    pltpu.sync_copy(agg_sh.at[pl.ds(row0, ROWS_PT)],
                    out_hbm.at[cid, pl.ds(row0, ROWS_PT)])


def kernel(X_n, edge_index, edge_attr, PE, snorm, batch, sketch_features,
           params):
    del snorm, sketch_features
    f32 = _f32
    xp = jnp.pad(X_n, ((0, NPAD - N), (0, 0)))
    pep = jnp.pad(PE, ((0, NPAD - N), (0, D - PE.shape[1])))
    w_pe_p = jnp.pad(params['W_pe'], ((0, D - PE.shape[1]), (0, 0)))

    src = jnp.pad(edge_index[0], (0, EPAD - E), constant_values=N)
    attr = jnp.pad(edge_attr, (0, EPAD - E))
    dst = jnp.pad(edge_index[1], (0, EPAD - E))
    edges = jnp.stack([src.reshape(-1, K), attr.reshape(-1, K),
                       dst.reshape(-1, K)], axis=1)
    zeros640 = jnp.zeros((ROWS_PT, D), f32)
    batch3 = jnp.pad(batch, (0, NPAD - N),
                     constant_values=jnp.int32(2 ** 30)).reshape(NBLK, 1, BLK)

    layers = params['layers']
    h, aug = _encoder(xp, pep, params['W_in'], w_pe_p, params['b_in'],
                      params['b_pe'], layers[0]['edge_emb'])

    for l in range(3):
        lp = layers[l]
        eps1 = jnp.reshape(lp['eps'], (1,))
        agg2 = _mp_kernel(aug.reshape(NA * NPAD, D), edges, zeros640)
        if l < 2:
            h, aug = _layer(h, agg2, eps1, lp['W1'], lp['b1'], lp['W2'],
                            lp['b2'], layers[l + 1]['edge_emb'])
        else:
            wf2p = jnp.pad(params['Wf2'], ((0, 0), (0, D - 1)))
            y = _final(h, agg2, eps1, lp['W1'], lp['b1'], lp['W2'], lp['b2'],
                       batch3, params['Wf1'], params['bf1'], wf2p,
                       params['bf2'])
    return y[:, 0]
